# baseline (speedup 1.0000x reference)
import functools

import jax
import jax.numpy as jnp
from jax import lax
from jax.experimental import pallas as pl
from jax.experimental.pallas import tpu as pltpu
from jax.experimental.pallas import tpu_sc as plsc

DIM = 16
LW = 128
N_NODES = 10000
N_EDGES = 160000
N_GRAPHS = 256
N_STEM = 20000
N_JBOND = 5000

N_TBL = 10240
NW = 32
EPW = 5120
E_PAD = NW * EPW
KCH = EPW // 128
N_PAD = 10240
NPW = N_PAD // 16
S_TOT = N_STEM + 2 * N_JBOND
S_PAD = 32768
SPW = S_PAD // NW
SKCH = SPW // 128


@functools.lru_cache(maxsize=1)
def _vec_mesh():
    return plsc.VectorSubcoreMesh(core_axis_name="c", subcore_axis_name="s")


def _lrelu(v):
    return jnp.where(v >= 0, v, 0.01 * v)


def _pad128(v, blk):
    return jnp.concatenate(
        [v, jnp.zeros((blk, LW - v.shape[1]), jnp.float32)], axis=1)


def _sc_gather(table, idx3):
    k_ch = idx3.shape[1]
    rows_w = k_ch * 128
    stg = N_TBL // 16

    @functools.partial(
        pl.kernel,
        mesh=_vec_mesh(),
        out_type=jax.ShapeDtypeStruct((NW * rows_w, LW), jnp.float32),
        scratch_types=[
            pltpu.VMEM((k_ch, 128), jnp.int32),
            pltpu.VMEM((128, LW), jnp.float32),
            pltpu.VMEM((128, LW), jnp.float32),
            pltpu.VMEM_SHARED((N_TBL, LW), jnp.float32),
            pltpu.SemaphoreType.DMA,
            pltpu.SemaphoreType.DMA,
        ],
    )
    def k(table_hbm, idx_hbm, out_hbm, idx_v, g0, g1, tbl_sh, w0, w1):
        c = lax.axis_index("c")
        s = lax.axis_index("s")
        wid = s * 2 + c
        base = wid * rows_w
        pltpu.sync_copy(table_hbm.at[pl.ds(s * stg, stg)],
                        tbl_sh.at[pl.ds(s * stg, stg)])
        pltpu.sync_copy(idx_hbm.at[wid], idx_v)
        plsc.subcore_barrier()
        pltpu.sync_copy(tbl_sh.at[idx_v.at[0]], g0)
        pltpu.async_copy(g0, out_hbm.at[pl.ds(base, 128)], w0)
        pltpu.sync_copy(tbl_sh.at[idx_v.at[1]], g1)
        pltpu.async_copy(g1, out_hbm.at[pl.ds(base + 128, 128)], w1)

        @pl.loop(2, k_ch, step=2)
        def _(j):
            pltpu.make_async_copy(g0, out_hbm.at[pl.ds(base, 128)], w0).wait()
            pltpu.sync_copy(tbl_sh.at[idx_v.at[j]], g0)
            pltpu.async_copy(g0, out_hbm.at[pl.ds(base + j * 128, 128)], w0)
            pltpu.make_async_copy(g1, out_hbm.at[pl.ds(base, 128)], w1).wait()
            pltpu.sync_copy(tbl_sh.at[idx_v.at[j + 1]], g1)
            pltpu.async_copy(
                g1, out_hbm.at[pl.ds(base + (j + 1) * 128, 128)], w1)

        pltpu.make_async_copy(g0, out_hbm.at[pl.ds(base, 128)], w0).wait()
        pltpu.make_async_copy(g1, out_hbm.at[pl.ds(base, 128)], w1).wait()

    return k(table, idx3)


def _sc_scatter_add(data, idx3, zeros):
    @functools.partial(
        pl.kernel,
        mesh=_vec_mesh(),
        out_type=jax.ShapeDtypeStruct((2, N_PAD, LW), jnp.float32),
        scratch_types=[
            pltpu.VMEM((KCH, 128), jnp.int32),
            pltpu.VMEM((128, LW), jnp.float32),
            pltpu.VMEM((128, LW), jnp.float32),
            pltpu.VMEM_SHARED((N_PAD, LW), jnp.float32),
            pltpu.SemaphoreType.DMA,
            pltpu.SemaphoreType.DMA,
        ],
    )
    def k(data_hbm, idx_hbm, zeros_hbm, out_hbm, idx_v, d0, d1, acc_sh,
          s0, s1):
        c = lax.axis_index("c")
        s = lax.axis_index("s")
        wid = s * 2 + c
        base = wid * EPW
        pltpu.sync_copy(zeros_hbm.at[pl.ds(s * NPW, NPW)],
                        acc_sh.at[pl.ds(s * NPW, NPW)])
        plsc.subcore_barrier()
        pltpu.sync_copy(idx_hbm.at[wid], idx_v)
        pltpu.async_copy(data_hbm.at[pl.ds(base, 128)], d0, s0)
        pltpu.async_copy(data_hbm.at[pl.ds(base + 128, 128)], d1, s1)

        @pl.loop(0, KCH, step=2)
        def _(j):
            nxt0 = base + jnp.minimum(j + 2, KCH - 1) * 128
            nxt1 = base + jnp.minimum(j + 3, KCH - 1) * 128
            pltpu.make_async_copy(data_hbm.at[pl.ds(base, 128)], d0, s0).wait()
            pltpu.sync_copy(d0, acc_sh.at[idx_v.at[j]], add=True)
            pltpu.async_copy(data_hbm.at[pl.ds(nxt0, 128)], d0, s0)
            pltpu.make_async_copy(data_hbm.at[pl.ds(base, 128)], d1, s1).wait()
            pltpu.sync_copy(d1, acc_sh.at[idx_v.at[j + 1]], add=True)
            pltpu.async_copy(data_hbm.at[pl.ds(nxt1, 128)], d1, s1)

        pltpu.make_async_copy(data_hbm.at[pl.ds(base, 128)], d0, s0).wait()
        pltpu.make_async_copy(data_hbm.at[pl.ds(base, 128)], d1, s1).wait()
        plsc.subcore_barrier()
        pltpu.sync_copy(acc_sh.at[pl.ds(s * NPW, NPW)],
                        out_hbm.at[c].at[pl.ds(s * NPW, NPW)])

    return k(data, idx3, zeros)


def _sc_count(ones_blk, idx3, zeros):
    @functools.partial(
        pl.kernel,
        mesh=_vec_mesh(),
        out_type=jax.ShapeDtypeStruct((2, N_PAD, LW), jnp.float32),
        scratch_types=[
            pltpu.VMEM((KCH, 128), jnp.int32),
            pltpu.VMEM((128, LW), jnp.float32),
            pltpu.VMEM_SHARED((N_PAD, LW), jnp.float32),
        ],
    )
    def k(ones_hbm, idx_hbm, zeros_hbm, out_hbm, idx_v, data_v, acc_sh):
        c = lax.axis_index("c")
        s = lax.axis_index("s")
        wid = s * 2 + c
        pltpu.sync_copy(zeros_hbm.at[pl.ds(s * NPW, NPW)],
                        acc_sh.at[pl.ds(s * NPW, NPW)])
        plsc.subcore_barrier()
        pltpu.sync_copy(idx_hbm.at[wid], idx_v)
        pltpu.sync_copy(ones_hbm, data_v)

        @pl.loop(0, KCH)
        def _(j):
            pltpu.sync_copy(data_v, acc_sh.at[idx_v.at[j]], add=True)

        plsc.subcore_barrier()
        pltpu.sync_copy(acc_sh.at[pl.ds(s * NPW, NPW)],
                        out_hbm.at[c].at[pl.ds(s * NPW, NPW)])

    return k(ones_blk, idx3, zeros)


def _tc_lin0(x, w_t, b):
    def body(x_ref, w_ref, b_ref, o_ref):
        v = _lrelu(
            jnp.dot(x_ref[...], w_ref[...],
                    preferred_element_type=jnp.float32) + b_ref[...])
        o_ref[...] = _pad128(v, N_TBL)

    return pl.pallas_call(
        body,
        out_shape=jax.ShapeDtypeStruct((N_TBL, LW), jnp.float32),
    )(x, w_t, b)


def _tc_h1(ea, w_t, b):
    blk = 8192

    def body(a_ref, w_ref, b_ref, o_ref):
        o_ref[...] = _lrelu(
            jnp.dot(a_ref[...], w_ref[...],
                    preferred_element_type=jnp.float32) + b_ref[...])

    return pl.pallas_call(
        body,
        grid=(E_PAD // blk,),
        in_specs=[
            pl.BlockSpec((blk, 4), lambda i: (i, 0)),
            pl.BlockSpec((4, DIM), lambda i: (0, 0)),
            pl.BlockSpec((1, DIM), lambda i: (0, 0)),
        ],
        out_specs=pl.BlockSpec((blk, DIM), lambda i: (i, 0)),
        out_shape=jax.ShapeDtypeStruct((E_PAD, DIM), jnp.float32),
        compiler_params=pltpu.CompilerParams(
            dimension_semantics=("parallel",)),
    )(ea, w_t, b)


def _tc_edge(xs, h1, ghat, b0, rm, cm):
    blk = 4096

    def body(xs_ref, h1_ref, g_ref, b0_ref, r_ref, c_ref, o_ref):
        xs_b = xs_ref[...][:, 0:DIM].astype(jnp.bfloat16)
        y = jnp.dot(xs_b, g_ref[...].astype(jnp.bfloat16),
                    preferred_element_type=jnp.float32)
        p = jnp.dot(h1_ref[...].astype(jnp.bfloat16),
                    r_ref[...].astype(jnp.bfloat16),
                    preferred_element_type=jnp.float32) * y
        msg = (jnp.dot(p.astype(jnp.bfloat16),
                       c_ref[...].astype(jnp.bfloat16),
                       preferred_element_type=jnp.float32)
               + jnp.dot(xs_b, b0_ref[...].astype(jnp.bfloat16),
                         preferred_element_type=jnp.float32))
        o_ref[...] = _pad128(msg, blk)

    return pl.pallas_call(
        body,
        grid=(E_PAD // blk,),
        in_specs=[
            pl.BlockSpec((blk, LW), lambda i: (i, 0)),
            pl.BlockSpec((blk, DIM), lambda i: (i, 0)),
            pl.BlockSpec((DIM, 256), lambda i: (0, 0)),
            pl.BlockSpec((DIM, DIM), lambda i: (0, 0)),
            pl.BlockSpec((DIM, 256), lambda i: (0, 0)),
            pl.BlockSpec((256, DIM), lambda i: (0, 0)),
        ],
        out_specs=pl.BlockSpec((blk, LW), lambda i: (i, 0)),
        out_shape=jax.ShapeDtypeStruct((E_PAD, LW), jnp.float32),
        compiler_params=pltpu.CompilerParams(
            dimension_semantics=("parallel",)),
    )(xs, h1, ghat, b0, rm, cm)


def _tc_node(s, parts, cnts, root, conv_b, wih_t, bih, whh_t, bhh):
    blk = 2048

    def body(s_ref, p_ref, c_ref, root_ref, cb_ref, wih_ref, bih_ref,
             whh_ref, bhh_ref, o_ref):
        s_b = s_ref[...][:, 0:DIM]
        agg = ((p_ref[0][:, 0:DIM] + p_ref[1][:, 0:DIM])
               / jnp.maximum(c_ref[0][:, 0:DIM] + c_ref[1][:, 0:DIM], 1.0))
        m = _lrelu(
            jnp.dot(s_b, root_ref[...], preferred_element_type=jnp.float32)
            + agg + cb_ref[...])
        gi = jnp.dot(m, wih_ref[...],
                     preferred_element_type=jnp.float32) + bih_ref[...]
        gh = jnp.dot(s_b, whh_ref[...],
                     preferred_element_type=jnp.float32) + bhh_ref[...]
        r = jax.nn.sigmoid(gi[:, 0:16] + gh[:, 0:16])
        z = jax.nn.sigmoid(gi[:, 16:32] + gh[:, 16:32])
        n = jnp.tanh(gi[:, 32:48] + r * gh[:, 32:48])
        o_ref[...] = _pad128((1.0 - z) * n + z * s_b, blk)

    return pl.pallas_call(
        body,
        grid=(N_TBL // blk,),
        in_specs=[
            pl.BlockSpec((blk, LW), lambda i: (i, 0)),
            pl.BlockSpec((2, blk, LW), lambda i: (0, i, 0)),
            pl.BlockSpec((2, blk, LW), lambda i: (0, i, 0)),
            pl.BlockSpec((DIM, DIM), lambda i: (0, 0)),
            pl.BlockSpec((1, DIM), lambda i: (0, 0)),
            pl.BlockSpec((DIM, 48), lambda i: (0, 0)),
            pl.BlockSpec((1, 48), lambda i: (0, 0)),
            pl.BlockSpec((DIM, 48), lambda i: (0, 0)),
            pl.BlockSpec((1, 48), lambda i: (0, 0)),
        ],
        out_specs=pl.BlockSpec((blk, LW), lambda i: (i, 0)),
        out_shape=jax.ShapeDtypeStruct((N_TBL, LW), jnp.float32),
        compiler_params=pltpu.CompilerParams(
            dimension_semantics=("parallel",)),
    )(s, parts, cnts, root, conv_b, wih_t, bih, whh_t, bhh)


def _tc_stem(ss, w1_t, b1, w2_t, b2):
    blk = 2000

    def body(ss_ref, w1_ref, b1_ref, w2_ref, b2_ref, o_ref):
        h = _lrelu(
            jnp.dot(ss_ref[...][:, 0:DIM], w1_ref[...],
                    preferred_element_type=jnp.float32) + b1_ref[...])
        o_ref[...] = jnp.dot(h, w2_ref[...],
                             preferred_element_type=jnp.float32) + b2_ref[...]

    return pl.pallas_call(
        body,
        grid=(N_STEM // blk,),
        in_specs=[
            pl.BlockSpec((blk, LW), lambda i: (i, 0)),
            pl.BlockSpec((DIM, DIM), lambda i: (0, 0)),
            pl.BlockSpec((1, DIM), lambda i: (0, 0)),
            pl.BlockSpec((DIM, 105), lambda i: (0, 0)),
            pl.BlockSpec((1, 105), lambda i: (0, 0)),
        ],
        out_specs=pl.BlockSpec((blk, 105), lambda i: (i, 0)),
        out_shape=jax.ShapeDtypeStruct((N_STEM, 105), jnp.float32),
        compiler_params=pltpu.CompilerParams(
            dimension_semantics=("parallel",)),
    )(ss, w1_t, b1, w2_t, b2)


def _tc_jbond(jb0, jb1, w1_t, b1, w2_t, b2):
    def body(a_ref, b_ref, w1_ref, b1_ref, w2_ref, b2_ref, o_ref):
        h0 = _lrelu(
            jnp.dot(a_ref[...][:, 0:DIM], w1_ref[...],
                    preferred_element_type=jnp.float32) + b1_ref[...])
        h1 = _lrelu(
            jnp.dot(b_ref[...][:, 0:DIM], w1_ref[...],
                    preferred_element_type=jnp.float32) + b1_ref[...])
        p = (jnp.dot(h0, w2_ref[...], preferred_element_type=jnp.float32)
             + jnp.dot(h1, w2_ref[...], preferred_element_type=jnp.float32))
        o_ref[...] = 0.5 * p + b2_ref[...]

    return pl.pallas_call(
        body,
        out_shape=jax.ShapeDtypeStruct((N_JBOND, 1), jnp.float32),
    )(jb0, jb1, w1_t, b1, w2_t, b2)


def _tc_set2set(out_s, batch2, bih, bhh, out_w_t, out_b):
    def body(s_ref, b_ref, bih_ref, bhh_ref, ow_ref, ob_ref, o_ref):
        s_b = s_ref[...][:, 0:DIM]
        g = bih_ref[...] + bhh_ref[...]
        cc = jax.nn.sigmoid(g[:, 0:16]) * jnp.tanh(g[:, 32:48])
        q0 = jax.nn.sigmoid(g[:, 48:64]) * jnp.tanh(cc)
        e = jnp.sum(s_b * q0, axis=1, keepdims=True)
        a = jnp.exp(e - jnp.max(e))
        oh = (b_ref[...] == lax.broadcasted_iota(
            jnp.int32, (N_NODES, N_GRAPHS), 1)).astype(jnp.float32)
        w = oh * a
        ext = jnp.concatenate(
            [s_b, jnp.ones((N_NODES, 1), jnp.float32)], axis=1)
        rv = lax.dot_general(w, ext, (((0,), (0,)), ((), ())),
                             preferred_element_type=jnp.float32)
        rvec = rv[:, 0:16] / (rv[:, 16:17] + 1e-16)
        q_star = jnp.concatenate(
            [jnp.broadcast_to(q0, (N_GRAPHS, DIM)), rvec], axis=1)
        o_ref[...] = jnp.dot(q_star, ow_ref[...],
                             preferred_element_type=jnp.float32) + ob_ref[...]

    return pl.pallas_call(
        body,
        grid=(1,),
        in_specs=[
            pl.BlockSpec((N_NODES, LW), lambda i: (0, 0)),
            pl.BlockSpec((N_NODES, 1), lambda i: (0, 0)),
            pl.BlockSpec((1, 64), lambda i: (0, 0)),
            pl.BlockSpec((1, 64), lambda i: (0, 0)),
            pl.BlockSpec((32, 2), lambda i: (0, 0)),
            pl.BlockSpec((1, 2), lambda i: (0, 0)),
        ],
        out_specs=pl.BlockSpec((N_GRAPHS, 2), lambda i: (0, 0)),
        out_shape=jax.ShapeDtypeStruct((N_GRAPHS, 2), jnp.float32),
    )(out_s, batch2, bih, bhh, out_w_t, out_b)


def kernel(x, edge_attr, params, edge_index, stem_atmidx, jbond_atmidx, batch):
    p = params
    src, dst = edge_index[0], edge_index[1]

    e_extra = E_PAD - N_EDGES
    src3 = jnp.concatenate(
        [src, jnp.zeros((e_extra,), jnp.int32)]).reshape(NW, KCH, 128)
    dst3 = jnp.concatenate(
        [dst, jnp.full((e_extra,), N_NODES, jnp.int32)]).reshape(NW, KCH, 128)
    sidx3 = jnp.concatenate([
        stem_atmidx, jbond_atmidx[:, 0], jbond_atmidx[:, 1],
        jnp.zeros((S_PAD - S_TOT,), jnp.int32)]).reshape(NW, SKCH, 128)
    xp = jnp.concatenate(
        [x, jnp.zeros((N_TBL - N_NODES, x.shape[1]), jnp.float32)], axis=0)
    ones_blk = jnp.ones((128, LW), jnp.float32)
    zeros_n = jnp.zeros((N_PAD, LW), jnp.float32)
    batch2 = batch.reshape(N_NODES, 1)

    w2 = p['edge_W2']
    ghat = w2.reshape(16, 16, 16).transpose(0, 2, 1).reshape(16, 256)
    b0 = p['edge_b2'].reshape(16, 16)
    eye = jnp.eye(16, dtype=jnp.float32)
    rm = jnp.repeat(eye, 16, axis=1)
    cm = jnp.tile(eye, (16, 1))
    lin0_wt = p['lin0_W'].T
    lin0_b = p['lin0_b'].reshape(1, DIM)
    e_w1t = p['edge_W1'].T
    e_b1 = p['edge_b1'].reshape(1, DIM)
    root = p['conv_root']
    conv_b = p['conv_bias'].reshape(1, DIM)
    wih_t = p['gru_Wih'].T
    whh_t = p['gru_Whh'].T
    bih = p['gru_bih'].reshape(1, 48)
    bhh = p['gru_bhh'].reshape(1, 48)
    st_w1t = p['stem_W1'].T
    st_b1 = p['stem_b1'].reshape(1, DIM)
    st_w2t = p['stem_W2'].T
    st_b2 = p['stem_b2'].reshape(1, 105)
    jb_w1t = p['jbond_W1'].T
    jb_b1 = p['jbond_b1'].reshape(1, DIM)
    jb_w2t = p['jbond_W2'].T
    jb_b2 = p['jbond_b2'].reshape(1, 1)
    l_bih = p['lstm_bih'].reshape(1, 64)
    l_bhh = p['lstm_bhh'].reshape(1, 64)
    out_wt = p['out_W'].T
    out_b = p['out_b'].reshape(1, 2)

    s = _tc_lin0(xp, lin0_wt, lin0_b)
    h1 = _tc_h1(edge_attr, e_w1t, e_b1)
    cnts = _sc_count(ones_blk, dst3, zeros_n)

    for _ in range(6):
        xs = _sc_gather(s, src3)
        msg = _tc_edge(xs, h1, ghat, b0, rm, cm)
        parts = _sc_scatter_add(msg, dst3, zeros_n)
        s = _tc_node(s, parts, cnts, root, conv_b, wih_t, bih, whh_t, bhh)

    heads = _sc_gather(s, sidx3)
    ss = heads[:N_STEM]
    jb0 = heads[N_STEM:N_STEM + N_JBOND]
    jb1 = heads[N_STEM + N_JBOND:S_TOT]

    stem_preds = _tc_stem(ss, st_w1t, st_b1, st_w2t, st_b2)
    jbond_preds = _tc_jbond(jb0, jb1, jb_w1t, jb_b1, jb_w2t,
                            jb_b2).reshape(N_JBOND)
    res = _tc_set2set(s, batch2, l_bih, l_bhh, out_wt, out_b)
    return res, stem_preds, jbond_preds


# speedup vs baseline: 1.0480x; 1.0480x over previous
import functools

import jax
import jax.numpy as jnp
from jax import lax
from jax.experimental import pallas as pl
from jax.experimental.pallas import tpu as pltpu
from jax.experimental.pallas import tpu_sc as plsc

DIM = 16
LW = 128
N_NODES = 10000
N_EDGES = 160000
N_GRAPHS = 256
N_STEM = 20000
N_JBOND = 5000

N_TBL = 10240
NW = 32
EPW = 5120
E_PAD = NW * EPW
KCH = EPW // 128
N_PAD = 10240
NPW = N_PAD // 16
S_TOT = N_STEM + 2 * N_JBOND
S_PAD = 32768
SPW = S_PAD // NW
SKCH = SPW // 128


@functools.lru_cache(maxsize=1)
def _vec_mesh():
    return plsc.VectorSubcoreMesh(core_axis_name="c", subcore_axis_name="s")


def _lrelu(v):
    return jnp.where(v >= 0, v, 0.01 * v)


def _pad128(v, blk):
    return jnp.concatenate(
        [v, jnp.zeros((blk, LW - v.shape[1]), jnp.float32)], axis=1)


def _sc_gather(table, idx3):
    k_ch = idx3.shape[1]
    rows_w = k_ch * 128
    stg = N_TBL // 16

    @functools.partial(
        pl.kernel,
        mesh=_vec_mesh(),
        out_type=jax.ShapeDtypeStruct((NW * rows_w, LW), jnp.float32),
        scratch_types=[
            pltpu.VMEM((k_ch, 128), jnp.int32),
            pltpu.VMEM((128, LW), jnp.float32),
            pltpu.VMEM((128, LW), jnp.float32),
            pltpu.VMEM_SHARED((N_TBL, LW), jnp.float32),
            pltpu.SemaphoreType.DMA,
            pltpu.SemaphoreType.DMA,
        ],
    )
    def k(table_hbm, idx_hbm, out_hbm, idx_v, g0, g1, tbl_sh, w0, w1):
        c = lax.axis_index("c")
        s = lax.axis_index("s")
        wid = s * 2 + c
        base = wid * rows_w
        pltpu.sync_copy(table_hbm.at[pl.ds(s * stg, stg)],
                        tbl_sh.at[pl.ds(s * stg, stg)])
        pltpu.sync_copy(idx_hbm.at[wid], idx_v)
        plsc.subcore_barrier()
        pltpu.sync_copy(tbl_sh.at[idx_v.at[0]], g0)
        pltpu.async_copy(g0, out_hbm.at[pl.ds(base, 128)], w0)
        pltpu.sync_copy(tbl_sh.at[idx_v.at[1]], g1)
        pltpu.async_copy(g1, out_hbm.at[pl.ds(base + 128, 128)], w1)

        @pl.loop(2, k_ch, step=2)
        def _(j):
            pltpu.make_async_copy(g0, out_hbm.at[pl.ds(base, 128)], w0).wait()
            pltpu.sync_copy(tbl_sh.at[idx_v.at[j]], g0)
            pltpu.async_copy(g0, out_hbm.at[pl.ds(base + j * 128, 128)], w0)
            pltpu.make_async_copy(g1, out_hbm.at[pl.ds(base, 128)], w1).wait()
            pltpu.sync_copy(tbl_sh.at[idx_v.at[j + 1]], g1)
            pltpu.async_copy(
                g1, out_hbm.at[pl.ds(base + (j + 1) * 128, 128)], w1)

        pltpu.make_async_copy(g0, out_hbm.at[pl.ds(base, 128)], w0).wait()
        pltpu.make_async_copy(g1, out_hbm.at[pl.ds(base, 128)], w1).wait()

    return k(table, idx3)


def _sc_scatter_add(data, idx3, zeros):
    k_ch = idx3.shape[1]
    epw = k_ch * 128

    @functools.partial(
        pl.kernel,
        mesh=_vec_mesh(),
        out_type=jax.ShapeDtypeStruct((2, N_PAD, LW), jnp.float32),
        scratch_types=[
            pltpu.VMEM((k_ch, 128), jnp.int32),
            pltpu.VMEM((128, LW), jnp.float32),
            pltpu.VMEM((128, LW), jnp.float32),
            pltpu.VMEM_SHARED((N_PAD, LW), jnp.float32),
            pltpu.SemaphoreType.DMA,
            pltpu.SemaphoreType.DMA,
        ],
    )
    def k(data_hbm, idx_hbm, zeros_hbm, out_hbm, idx_v, d0, d1, acc_sh,
          s0, s1):
        c = lax.axis_index("c")
        s = lax.axis_index("s")
        wid = s * 2 + c
        base = wid * epw
        pltpu.sync_copy(zeros_hbm.at[pl.ds(s * NPW, NPW)],
                        acc_sh.at[pl.ds(s * NPW, NPW)])
        plsc.subcore_barrier()
        pltpu.sync_copy(idx_hbm.at[wid], idx_v)
        pltpu.async_copy(data_hbm.at[pl.ds(base, 128)], d0, s0)
        pltpu.async_copy(data_hbm.at[pl.ds(base + 128, 128)], d1, s1)

        @pl.loop(0, k_ch, step=2)
        def _(j):
            nxt0 = base + jnp.minimum(j + 2, k_ch - 1) * 128
            nxt1 = base + jnp.minimum(j + 3, k_ch - 1) * 128
            pltpu.make_async_copy(data_hbm.at[pl.ds(base, 128)], d0, s0).wait()
            pltpu.sync_copy(d0, acc_sh.at[idx_v.at[j]], add=True)
            pltpu.async_copy(data_hbm.at[pl.ds(nxt0, 128)], d0, s0)
            pltpu.make_async_copy(data_hbm.at[pl.ds(base, 128)], d1, s1).wait()
            pltpu.sync_copy(d1, acc_sh.at[idx_v.at[j + 1]], add=True)
            pltpu.async_copy(data_hbm.at[pl.ds(nxt1, 128)], d1, s1)

        pltpu.make_async_copy(data_hbm.at[pl.ds(base, 128)], d0, s0).wait()
        pltpu.make_async_copy(data_hbm.at[pl.ds(base, 128)], d1, s1).wait()
        plsc.subcore_barrier()
        pltpu.sync_copy(acc_sh.at[pl.ds(s * NPW, NPW)],
                        out_hbm.at[c].at[pl.ds(s * NPW, NPW)])

    return k(data, idx3, zeros)


def _sc_count(ones_blk, idx3, zeros):
    @functools.partial(
        pl.kernel,
        mesh=_vec_mesh(),
        out_type=jax.ShapeDtypeStruct((2, N_PAD, LW), jnp.float32),
        scratch_types=[
            pltpu.VMEM((KCH, 128), jnp.int32),
            pltpu.VMEM((128, LW), jnp.float32),
            pltpu.VMEM_SHARED((N_PAD, LW), jnp.float32),
        ],
    )
    def k(ones_hbm, idx_hbm, zeros_hbm, out_hbm, idx_v, data_v, acc_sh):
        c = lax.axis_index("c")
        s = lax.axis_index("s")
        wid = s * 2 + c
        pltpu.sync_copy(zeros_hbm.at[pl.ds(s * NPW, NPW)],
                        acc_sh.at[pl.ds(s * NPW, NPW)])
        plsc.subcore_barrier()
        pltpu.sync_copy(idx_hbm.at[wid], idx_v)
        pltpu.sync_copy(ones_hbm, data_v)

        @pl.loop(0, KCH)
        def _(j):
            pltpu.sync_copy(data_v, acc_sh.at[idx_v.at[j]], add=True)

        plsc.subcore_barrier()
        pltpu.sync_copy(acc_sh.at[pl.ds(s * NPW, NPW)],
                        out_hbm.at[c].at[pl.ds(s * NPW, NPW)])

    return k(ones_blk, idx3, zeros)


def _tc_lin0(x, w_t, b):
    def body(x_ref, w_ref, b_ref, o_ref):
        v = _lrelu(
            jnp.dot(x_ref[...], w_ref[...],
                    preferred_element_type=jnp.float32) + b_ref[...])
        o_ref[...] = _pad128(v, N_TBL)

    return pl.pallas_call(
        body,
        out_shape=jax.ShapeDtypeStruct((N_TBL, LW), jnp.float32),
    )(x, w_t, b)


def _tc_h1(ea, w_t, b):
    blk = 8192

    def body(a_ref, w_ref, b_ref, o_ref):
        o_ref[...] = _lrelu(
            jnp.dot(a_ref[...], w_ref[...],
                    preferred_element_type=jnp.float32) + b_ref[...])

    return pl.pallas_call(
        body,
        grid=(E_PAD // blk,),
        in_specs=[
            pl.BlockSpec((blk, 4), lambda i: (i, 0)),
            pl.BlockSpec((4, DIM), lambda i: (0, 0)),
            pl.BlockSpec((1, DIM), lambda i: (0, 0)),
        ],
        out_specs=pl.BlockSpec((blk, DIM), lambda i: (i, 0)),
        out_shape=jax.ShapeDtypeStruct((E_PAD, DIM), jnp.float32),
        compiler_params=pltpu.CompilerParams(
            dimension_semantics=("parallel",)),
    )(ea, w_t, b)


def _tc_edge(xs, h1, ghat, b0, rm, cm, h1_off):
    blk = 4096
    n_blk = xs.shape[0] // blk

    def body(xs_ref, h1_ref, g_ref, b0_ref, r_ref, c_ref, o_ref):
        xs_b = xs_ref[...][:, 0:DIM].astype(jnp.bfloat16)
        y = jnp.dot(xs_b, g_ref[...].astype(jnp.bfloat16),
                    preferred_element_type=jnp.float32)
        p = jnp.dot(h1_ref[...].astype(jnp.bfloat16),
                    r_ref[...].astype(jnp.bfloat16),
                    preferred_element_type=jnp.float32) * y
        msg = (jnp.dot(p.astype(jnp.bfloat16),
                       c_ref[...].astype(jnp.bfloat16),
                       preferred_element_type=jnp.float32)
               + jnp.dot(xs_b, b0_ref[...].astype(jnp.bfloat16),
                         preferred_element_type=jnp.float32))
        o_ref[...] = _pad128(msg, blk)

    return pl.pallas_call(
        body,
        grid=(n_blk,),
        in_specs=[
            pl.BlockSpec((blk, LW), lambda i: (i, 0)),
            pl.BlockSpec((blk, DIM), lambda i: (i + h1_off, 0)),
            pl.BlockSpec((DIM, 256), lambda i: (0, 0)),
            pl.BlockSpec((DIM, DIM), lambda i: (0, 0)),
            pl.BlockSpec((DIM, 256), lambda i: (0, 0)),
            pl.BlockSpec((256, DIM), lambda i: (0, 0)),
        ],
        out_specs=pl.BlockSpec((blk, LW), lambda i: (i, 0)),
        out_shape=jax.ShapeDtypeStruct((xs.shape[0], LW), jnp.float32),
        compiler_params=pltpu.CompilerParams(
            dimension_semantics=("parallel",)),
    )(xs, h1, ghat, b0, rm, cm)


def _tc_node(s, parts_a, parts_b, cnts, root, conv_b, wih_t, bih, whh_t, bhh):
    blk = 2048

    def body(s_ref, p_ref, q_ref, c_ref, root_ref, cb_ref, wih_ref, bih_ref,
             whh_ref, bhh_ref, o_ref):
        s_b = s_ref[...][:, 0:DIM]
        agg = ((p_ref[0][:, 0:DIM] + p_ref[1][:, 0:DIM]
                + q_ref[0][:, 0:DIM] + q_ref[1][:, 0:DIM])
               / jnp.maximum(c_ref[0][:, 0:DIM] + c_ref[1][:, 0:DIM], 1.0))
        m = _lrelu(
            jnp.dot(s_b, root_ref[...], preferred_element_type=jnp.float32)
            + agg + cb_ref[...])
        gi = jnp.dot(m, wih_ref[...],
                     preferred_element_type=jnp.float32) + bih_ref[...]
        gh = jnp.dot(s_b, whh_ref[...],
                     preferred_element_type=jnp.float32) + bhh_ref[...]
        r = jax.nn.sigmoid(gi[:, 0:16] + gh[:, 0:16])
        z = jax.nn.sigmoid(gi[:, 16:32] + gh[:, 16:32])
        n = jnp.tanh(gi[:, 32:48] + r * gh[:, 32:48])
        o_ref[...] = _pad128((1.0 - z) * n + z * s_b, blk)

    return pl.pallas_call(
        body,
        grid=(N_TBL // blk,),
        in_specs=[
            pl.BlockSpec((blk, LW), lambda i: (i, 0)),
            pl.BlockSpec((2, blk, LW), lambda i: (0, i, 0)),
            pl.BlockSpec((2, blk, LW), lambda i: (0, i, 0)),
            pl.BlockSpec((2, blk, LW), lambda i: (0, i, 0)),
            pl.BlockSpec((DIM, DIM), lambda i: (0, 0)),
            pl.BlockSpec((1, DIM), lambda i: (0, 0)),
            pl.BlockSpec((DIM, 48), lambda i: (0, 0)),
            pl.BlockSpec((1, 48), lambda i: (0, 0)),
            pl.BlockSpec((DIM, 48), lambda i: (0, 0)),
            pl.BlockSpec((1, 48), lambda i: (0, 0)),
        ],
        out_specs=pl.BlockSpec((blk, LW), lambda i: (i, 0)),
        out_shape=jax.ShapeDtypeStruct((N_TBL, LW), jnp.float32),
        compiler_params=pltpu.CompilerParams(
            dimension_semantics=("parallel",)),
    )(s, parts_a, parts_b, cnts, root, conv_b, wih_t, bih, whh_t, bhh)


def _tc_stem(ss, w1_t, b1, w2_t, b2):
    blk = 2000

    def body(ss_ref, w1_ref, b1_ref, w2_ref, b2_ref, o_ref):
        h = _lrelu(
            jnp.dot(ss_ref[...][:, 0:DIM], w1_ref[...],
                    preferred_element_type=jnp.float32) + b1_ref[...])
        o_ref[...] = jnp.dot(h, w2_ref[...],
                             preferred_element_type=jnp.float32) + b2_ref[...]

    return pl.pallas_call(
        body,
        grid=(N_STEM // blk,),
        in_specs=[
            pl.BlockSpec((blk, LW), lambda i: (i, 0)),
            pl.BlockSpec((DIM, DIM), lambda i: (0, 0)),
            pl.BlockSpec((1, DIM), lambda i: (0, 0)),
            pl.BlockSpec((DIM, 105), lambda i: (0, 0)),
            pl.BlockSpec((1, 105), lambda i: (0, 0)),
        ],
        out_specs=pl.BlockSpec((blk, 105), lambda i: (i, 0)),
        out_shape=jax.ShapeDtypeStruct((N_STEM, 105), jnp.float32),
        compiler_params=pltpu.CompilerParams(
            dimension_semantics=("parallel",)),
    )(ss, w1_t, b1, w2_t, b2)


def _tc_jbond(jb0, jb1, w1_t, b1, w2_t, b2):
    def body(a_ref, b_ref, w1_ref, b1_ref, w2_ref, b2_ref, o_ref):
        h0 = _lrelu(
            jnp.dot(a_ref[...][:, 0:DIM], w1_ref[...],
                    preferred_element_type=jnp.float32) + b1_ref[...])
        h1 = _lrelu(
            jnp.dot(b_ref[...][:, 0:DIM], w1_ref[...],
                    preferred_element_type=jnp.float32) + b1_ref[...])
        p = (jnp.dot(h0, w2_ref[...], preferred_element_type=jnp.float32)
             + jnp.dot(h1, w2_ref[...], preferred_element_type=jnp.float32))
        o_ref[...] = 0.5 * p + b2_ref[...]

    return pl.pallas_call(
        body,
        out_shape=jax.ShapeDtypeStruct((N_JBOND, 1), jnp.float32),
    )(jb0, jb1, w1_t, b1, w2_t, b2)


def _tc_set2set(out_s, batch2, bih, bhh, out_w_t, out_b):
    def body(s_ref, b_ref, bih_ref, bhh_ref, ow_ref, ob_ref, o_ref):
        s_b = s_ref[...][:, 0:DIM]
        g = bih_ref[...] + bhh_ref[...]
        cc = jax.nn.sigmoid(g[:, 0:16]) * jnp.tanh(g[:, 32:48])
        q0 = jax.nn.sigmoid(g[:, 48:64]) * jnp.tanh(cc)
        e = jnp.sum(s_b * q0, axis=1, keepdims=True)
        a = jnp.exp(e - jnp.max(e))
        oh = (b_ref[...] == lax.broadcasted_iota(
            jnp.int32, (N_NODES, N_GRAPHS), 1)).astype(jnp.float32)
        w = oh * a
        ext = jnp.concatenate(
            [s_b, jnp.ones((N_NODES, 1), jnp.float32)], axis=1)
        rv = lax.dot_general(w, ext, (((0,), (0,)), ((), ())),
                             preferred_element_type=jnp.float32)
        rvec = rv[:, 0:16] / (rv[:, 16:17] + 1e-16)
        q_star = jnp.concatenate(
            [jnp.broadcast_to(q0, (N_GRAPHS, DIM)), rvec], axis=1)
        o_ref[...] = jnp.dot(q_star, ow_ref[...],
                             preferred_element_type=jnp.float32) + ob_ref[...]

    return pl.pallas_call(
        body,
        grid=(1,),
        in_specs=[
            pl.BlockSpec((N_NODES, LW), lambda i: (0, 0)),
            pl.BlockSpec((N_NODES, 1), lambda i: (0, 0)),
            pl.BlockSpec((1, 64), lambda i: (0, 0)),
            pl.BlockSpec((1, 64), lambda i: (0, 0)),
            pl.BlockSpec((32, 2), lambda i: (0, 0)),
            pl.BlockSpec((1, 2), lambda i: (0, 0)),
        ],
        out_specs=pl.BlockSpec((N_GRAPHS, 2), lambda i: (0, 0)),
        out_shape=jax.ShapeDtypeStruct((N_GRAPHS, 2), jnp.float32),
    )(out_s, batch2, bih, bhh, out_w_t, out_b)


def kernel(x, edge_attr, params, edge_index, stem_atmidx, jbond_atmidx, batch):
    p = params
    src, dst = edge_index[0], edge_index[1]

    e_extra = E_PAD - N_EDGES
    srcp = jnp.concatenate([src, jnp.zeros((e_extra,), jnp.int32)])
    dstp = jnp.concatenate([dst, jnp.full((e_extra,), N_NODES, jnp.int32)])
    eh = E_PAD // 2
    src3a = srcp[:eh].reshape(NW, KCH // 2, 128)
    src3b = srcp[eh:].reshape(NW, KCH // 2, 128)
    dst3a = dstp[:eh].reshape(NW, KCH // 2, 128)
    dst3b = dstp[eh:].reshape(NW, KCH // 2, 128)
    dst3 = dstp.reshape(NW, KCH, 128)
    sidx3 = jnp.concatenate([
        stem_atmidx, jbond_atmidx[:, 0], jbond_atmidx[:, 1],
        jnp.zeros((S_PAD - S_TOT,), jnp.int32)]).reshape(NW, SKCH, 128)
    xp = jnp.concatenate(
        [x, jnp.zeros((N_TBL - N_NODES, x.shape[1]), jnp.float32)], axis=0)
    ones_blk = jnp.ones((128, LW), jnp.float32)
    zeros_n = jnp.zeros((N_PAD, LW), jnp.float32)
    batch2 = batch.reshape(N_NODES, 1)

    w2 = p['edge_W2']
    ghat = w2.reshape(16, 16, 16).transpose(0, 2, 1).reshape(16, 256)
    b0 = p['edge_b2'].reshape(16, 16)
    eye = jnp.eye(16, dtype=jnp.float32)
    rm = jnp.repeat(eye, 16, axis=1)
    cm = jnp.tile(eye, (16, 1))
    lin0_wt = p['lin0_W'].T
    lin0_b = p['lin0_b'].reshape(1, DIM)
    e_w1t = p['edge_W1'].T
    e_b1 = p['edge_b1'].reshape(1, DIM)
    root = p['conv_root']
    conv_b = p['conv_bias'].reshape(1, DIM)
    wih_t = p['gru_Wih'].T
    whh_t = p['gru_Whh'].T
    bih = p['gru_bih'].reshape(1, 48)
    bhh = p['gru_bhh'].reshape(1, 48)
    st_w1t = p['stem_W1'].T
    st_b1 = p['stem_b1'].reshape(1, DIM)
    st_w2t = p['stem_W2'].T
    st_b2 = p['stem_b2'].reshape(1, 105)
    jb_w1t = p['jbond_W1'].T
    jb_b1 = p['jbond_b1'].reshape(1, DIM)
    jb_w2t = p['jbond_W2'].T
    jb_b2 = p['jbond_b2'].reshape(1, 1)
    l_bih = p['lstm_bih'].reshape(1, 64)
    l_bhh = p['lstm_bhh'].reshape(1, 64)
    out_wt = p['out_W'].T
    out_b = p['out_b'].reshape(1, 2)

    s = _tc_lin0(xp, lin0_wt, lin0_b)
    h1 = _tc_h1(edge_attr, e_w1t, e_b1)
    cnts = _sc_count(ones_blk, dst3, zeros_n)

    h1_half_blocks = eh // 4096
    for _ in range(6):
        xs_a = _sc_gather(s, src3a)
        msg_a = _tc_edge(xs_a, h1, ghat, b0, rm, cm, 0)
        xs_b = _sc_gather(s, src3b)
        parts_a = _sc_scatter_add(msg_a, dst3a, zeros_n)
        msg_b = _tc_edge(xs_b, h1, ghat, b0, rm, cm, h1_half_blocks)
        parts_b = _sc_scatter_add(msg_b, dst3b, zeros_n)
        s = _tc_node(s, parts_a, parts_b, cnts, root, conv_b, wih_t,
                     bih, whh_t, bhh)

    heads = _sc_gather(s, sidx3)
    ss = heads[:N_STEM]
    jb0 = heads[N_STEM:N_STEM + N_JBOND]
    jb1 = heads[N_STEM + N_JBOND:S_TOT]

    stem_preds = _tc_stem(ss, st_w1t, st_b1, st_w2t, st_b2)
    jbond_preds = _tc_jbond(jb0, jb1, jb_w1t, jb_b1, jb_w2t,
                            jb_b2).reshape(N_JBOND)
    res = _tc_set2set(s, batch2, l_bih, l_bhh, out_wt, out_b)
    return res, stem_preds, jbond_preds


# speedup vs baseline: 1.0494x; 1.0014x over previous
import functools

import jax
import jax.numpy as jnp
from jax import lax
from jax.experimental import pallas as pl
from jax.experimental.pallas import tpu as pltpu
from jax.experimental.pallas import tpu_sc as plsc

DIM = 16
LW = 128
N_NODES = 10000
N_EDGES = 160000
N_GRAPHS = 256
N_STEM = 20000
N_JBOND = 5000

N_TBL = 10240
NW = 32
EPW = 5120
E_PAD = NW * EPW
KCH = EPW // 128
N_PAD = 10240
NPW = N_PAD // 16
S_TOT = N_STEM + 2 * N_JBOND
S_PAD = 32768
SPW = S_PAD // NW
SKCH = SPW // 128


@functools.lru_cache(maxsize=1)
def _vec_mesh():
    return plsc.VectorSubcoreMesh(core_axis_name="c", subcore_axis_name="s")


def _lrelu(v):
    return jnp.where(v >= 0, v, 0.01 * v)


def _pad128(v, blk):
    return jnp.concatenate(
        [v, jnp.zeros((blk, LW - v.shape[1]), jnp.float32)], axis=1)


def _sc_gather(table, idx3):
    k_ch = idx3.shape[1]
    rows_w = k_ch * 128
    stg = N_TBL // 16

    @functools.partial(
        pl.kernel,
        mesh=_vec_mesh(),
        out_type=jax.ShapeDtypeStruct((NW * rows_w, LW), jnp.float32),
        scratch_types=[
            pltpu.VMEM((k_ch, 128), jnp.int32),
            pltpu.VMEM((128, LW), jnp.float32),
            pltpu.VMEM((128, LW), jnp.float32),
            pltpu.VMEM_SHARED((N_TBL, LW), jnp.float32),
            pltpu.SemaphoreType.DMA,
            pltpu.SemaphoreType.DMA,
        ],
    )
    def k(table_hbm, idx_hbm, out_hbm, idx_v, g0, g1, tbl_sh, w0, w1):
        c = lax.axis_index("c")
        s = lax.axis_index("s")
        wid = s * 2 + c
        base = wid * rows_w
        pltpu.sync_copy(table_hbm.at[pl.ds(s * stg, stg)],
                        tbl_sh.at[pl.ds(s * stg, stg)])
        pltpu.sync_copy(idx_hbm.at[wid], idx_v)
        plsc.subcore_barrier()
        pltpu.sync_copy(tbl_sh.at[idx_v.at[0]], g0)
        pltpu.async_copy(g0, out_hbm.at[pl.ds(base, 128)], w0)
        pltpu.sync_copy(tbl_sh.at[idx_v.at[1]], g1)
        pltpu.async_copy(g1, out_hbm.at[pl.ds(base + 128, 128)], w1)

        @pl.loop(2, k_ch, step=2)
        def _(j):
            pltpu.make_async_copy(g0, out_hbm.at[pl.ds(base, 128)], w0).wait()
            pltpu.sync_copy(tbl_sh.at[idx_v.at[j]], g0)
            pltpu.async_copy(g0, out_hbm.at[pl.ds(base + j * 128, 128)], w0)
            pltpu.make_async_copy(g1, out_hbm.at[pl.ds(base, 128)], w1).wait()
            pltpu.sync_copy(tbl_sh.at[idx_v.at[j + 1]], g1)
            pltpu.async_copy(
                g1, out_hbm.at[pl.ds(base + (j + 1) * 128, 128)], w1)

        pltpu.make_async_copy(g0, out_hbm.at[pl.ds(base, 128)], w0).wait()
        pltpu.make_async_copy(g1, out_hbm.at[pl.ds(base, 128)], w1).wait()

    return k(table, idx3)


def _sc_scatter_add(data, idx3, zeros):
    k_ch = idx3.shape[1]
    epw = k_ch * 128

    @functools.partial(
        pl.kernel,
        mesh=_vec_mesh(),
        out_type=jax.ShapeDtypeStruct((2, N_PAD, LW), jnp.float32),
        scratch_types=[
            pltpu.VMEM((k_ch, 128), jnp.int32),
            pltpu.VMEM((128, LW), jnp.float32),
            pltpu.VMEM((128, LW), jnp.float32),
            pltpu.VMEM_SHARED((N_PAD, LW), jnp.float32),
            pltpu.SemaphoreType.DMA,
            pltpu.SemaphoreType.DMA,
        ],
    )
    def k(data_hbm, idx_hbm, zeros_hbm, out_hbm, idx_v, d0, d1, acc_sh,
          s0, s1):
        c = lax.axis_index("c")
        s = lax.axis_index("s")
        wid = s * 2 + c
        base = wid * epw
        pltpu.sync_copy(zeros_hbm.at[pl.ds(s * NPW, NPW)],
                        acc_sh.at[pl.ds(s * NPW, NPW)])
        plsc.subcore_barrier()
        pltpu.sync_copy(idx_hbm.at[wid], idx_v)
        pltpu.async_copy(data_hbm.at[pl.ds(base, 128)], d0, s0)
        pltpu.async_copy(data_hbm.at[pl.ds(base + 128, 128)], d1, s1)

        @pl.loop(0, k_ch, step=2)
        def _(j):
            nxt0 = base + jnp.minimum(j + 2, k_ch - 1) * 128
            nxt1 = base + jnp.minimum(j + 3, k_ch - 1) * 128
            pltpu.make_async_copy(data_hbm.at[pl.ds(base, 128)], d0, s0).wait()
            pltpu.sync_copy(d0, acc_sh.at[idx_v.at[j]], add=True)
            pltpu.async_copy(data_hbm.at[pl.ds(nxt0, 128)], d0, s0)
            pltpu.make_async_copy(data_hbm.at[pl.ds(base, 128)], d1, s1).wait()
            pltpu.sync_copy(d1, acc_sh.at[idx_v.at[j + 1]], add=True)
            pltpu.async_copy(data_hbm.at[pl.ds(nxt1, 128)], d1, s1)

        pltpu.make_async_copy(data_hbm.at[pl.ds(base, 128)], d0, s0).wait()
        pltpu.make_async_copy(data_hbm.at[pl.ds(base, 128)], d1, s1).wait()
        plsc.subcore_barrier()
        pltpu.sync_copy(acc_sh.at[pl.ds(s * NPW, NPW)],
                        out_hbm.at[c].at[pl.ds(s * NPW, NPW)])

    return k(data, idx3, zeros)


def _sc_count(ones_blk, idx3, zeros):
    @functools.partial(
        pl.kernel,
        mesh=_vec_mesh(),
        out_type=jax.ShapeDtypeStruct((2, N_PAD, LW), jnp.float32),
        scratch_types=[
            pltpu.VMEM((KCH, 128), jnp.int32),
            pltpu.VMEM((128, LW), jnp.float32),
            pltpu.VMEM_SHARED((N_PAD, LW), jnp.float32),
        ],
    )
    def k(ones_hbm, idx_hbm, zeros_hbm, out_hbm, idx_v, data_v, acc_sh):
        c = lax.axis_index("c")
        s = lax.axis_index("s")
        wid = s * 2 + c
        pltpu.sync_copy(zeros_hbm.at[pl.ds(s * NPW, NPW)],
                        acc_sh.at[pl.ds(s * NPW, NPW)])
        plsc.subcore_barrier()
        pltpu.sync_copy(idx_hbm.at[wid], idx_v)
        pltpu.sync_copy(ones_hbm, data_v)

        @pl.loop(0, KCH)
        def _(j):
            pltpu.sync_copy(data_v, acc_sh.at[idx_v.at[j]], add=True)

        plsc.subcore_barrier()
        pltpu.sync_copy(acc_sh.at[pl.ds(s * NPW, NPW)],
                        out_hbm.at[c].at[pl.ds(s * NPW, NPW)])

    return k(ones_blk, idx3, zeros)


def _tc_lin0(x, w_t, b):
    def body(x_ref, w_ref, b_ref, o_ref):
        v = _lrelu(
            jnp.dot(x_ref[...], w_ref[...],
                    preferred_element_type=jnp.float32) + b_ref[...])
        o_ref[...] = _pad128(v, N_TBL)

    return pl.pallas_call(
        body,
        out_shape=jax.ShapeDtypeStruct((N_TBL, LW), jnp.float32),
    )(x, w_t, b)


def _tc_h1(ea, w_t, b):
    blk = 8192

    def body(a_ref, w_ref, b_ref, o_ref):
        o_ref[...] = _lrelu(
            jnp.dot(a_ref[...], w_ref[...],
                    preferred_element_type=jnp.float32) + b_ref[...])

    return pl.pallas_call(
        body,
        grid=(E_PAD // blk,),
        in_specs=[
            pl.BlockSpec((blk, 4), lambda i: (i, 0)),
            pl.BlockSpec((4, DIM), lambda i: (0, 0)),
            pl.BlockSpec((1, DIM), lambda i: (0, 0)),
        ],
        out_specs=pl.BlockSpec((blk, DIM), lambda i: (i, 0)),
        out_shape=jax.ShapeDtypeStruct((E_PAD, DIM), jnp.float32),
        compiler_params=pltpu.CompilerParams(
            dimension_semantics=("parallel",)),
    )(ea, w_t, b)


def _tc_edge(xs, h1, ghat, b0, rm, cm, h1_off):
    blk = 4096
    n_blk = xs.shape[0] // blk

    def body(xs_ref, h1_ref, g_ref, b0_ref, r_ref, c_ref, o_ref):
        xs_b = xs_ref[...][:, 0:DIM]
        y = jnp.dot(xs_b, g_ref[...], preferred_element_type=jnp.float32)
        p = jnp.dot(h1_ref[...], r_ref[...],
                    preferred_element_type=jnp.float32) * y
        msg = (jnp.dot(p, c_ref[...], preferred_element_type=jnp.float32)
               + jnp.dot(xs_b, b0_ref[...],
                         preferred_element_type=jnp.float32))
        o_ref[...] = _pad128(msg, blk)

    return pl.pallas_call(
        body,
        grid=(n_blk,),
        in_specs=[
            pl.BlockSpec((blk, LW), lambda i: (i, 0)),
            pl.BlockSpec((blk, DIM), lambda i: (i + h1_off, 0)),
            pl.BlockSpec((DIM, 256), lambda i: (0, 0)),
            pl.BlockSpec((DIM, DIM), lambda i: (0, 0)),
            pl.BlockSpec((DIM, 256), lambda i: (0, 0)),
            pl.BlockSpec((256, DIM), lambda i: (0, 0)),
        ],
        out_specs=pl.BlockSpec((blk, LW), lambda i: (i, 0)),
        out_shape=jax.ShapeDtypeStruct((xs.shape[0], LW), jnp.float32),
        compiler_params=pltpu.CompilerParams(
            dimension_semantics=("parallel",)),
    )(xs, h1, ghat, b0, rm, cm)


def _tc_node(s, parts_a, parts_b, cnts, root, conv_b, wih_t, bih, whh_t, bhh):
    blk = 2048

    def body(s_ref, p_ref, q_ref, c_ref, root_ref, cb_ref, wih_ref, bih_ref,
             whh_ref, bhh_ref, o_ref):
        s_b = s_ref[...][:, 0:DIM]
        agg = ((p_ref[0][:, 0:DIM] + p_ref[1][:, 0:DIM]
                + q_ref[0][:, 0:DIM] + q_ref[1][:, 0:DIM])
               / jnp.maximum(c_ref[0][:, 0:DIM] + c_ref[1][:, 0:DIM], 1.0))
        m = _lrelu(
            jnp.dot(s_b, root_ref[...], preferred_element_type=jnp.float32)
            + agg + cb_ref[...])
        gi = jnp.dot(m, wih_ref[...],
                     preferred_element_type=jnp.float32) + bih_ref[...]
        gh = jnp.dot(s_b, whh_ref[...],
                     preferred_element_type=jnp.float32) + bhh_ref[...]
        r = jax.nn.sigmoid(gi[:, 0:16] + gh[:, 0:16])
        z = jax.nn.sigmoid(gi[:, 16:32] + gh[:, 16:32])
        n = jnp.tanh(gi[:, 32:48] + r * gh[:, 32:48])
        o_ref[...] = _pad128((1.0 - z) * n + z * s_b, blk)

    return pl.pallas_call(
        body,
        grid=(N_TBL // blk,),
        in_specs=[
            pl.BlockSpec((blk, LW), lambda i: (i, 0)),
            pl.BlockSpec((2, blk, LW), lambda i: (0, i, 0)),
            pl.BlockSpec((2, blk, LW), lambda i: (0, i, 0)),
            pl.BlockSpec((2, blk, LW), lambda i: (0, i, 0)),
            pl.BlockSpec((DIM, DIM), lambda i: (0, 0)),
            pl.BlockSpec((1, DIM), lambda i: (0, 0)),
            pl.BlockSpec((DIM, 48), lambda i: (0, 0)),
            pl.BlockSpec((1, 48), lambda i: (0, 0)),
            pl.BlockSpec((DIM, 48), lambda i: (0, 0)),
            pl.BlockSpec((1, 48), lambda i: (0, 0)),
        ],
        out_specs=pl.BlockSpec((blk, LW), lambda i: (i, 0)),
        out_shape=jax.ShapeDtypeStruct((N_TBL, LW), jnp.float32),
        compiler_params=pltpu.CompilerParams(
            dimension_semantics=("parallel",)),
    )(s, parts_a, parts_b, cnts, root, conv_b, wih_t, bih, whh_t, bhh)


def _tc_stem(ss, w1_t, b1, w2_t, b2):
    blk = 2000

    def body(ss_ref, w1_ref, b1_ref, w2_ref, b2_ref, o_ref):
        h = _lrelu(
            jnp.dot(ss_ref[...][:, 0:DIM], w1_ref[...],
                    preferred_element_type=jnp.float32) + b1_ref[...])
        o_ref[...] = jnp.dot(h, w2_ref[...],
                             preferred_element_type=jnp.float32) + b2_ref[...]

    return pl.pallas_call(
        body,
        grid=(N_STEM // blk,),
        in_specs=[
            pl.BlockSpec((blk, LW), lambda i: (i, 0)),
            pl.BlockSpec((DIM, DIM), lambda i: (0, 0)),
            pl.BlockSpec((1, DIM), lambda i: (0, 0)),
            pl.BlockSpec((DIM, 105), lambda i: (0, 0)),
            pl.BlockSpec((1, 105), lambda i: (0, 0)),
        ],
        out_specs=pl.BlockSpec((blk, 105), lambda i: (i, 0)),
        out_shape=jax.ShapeDtypeStruct((N_STEM, 105), jnp.float32),
        compiler_params=pltpu.CompilerParams(
            dimension_semantics=("parallel",)),
    )(ss, w1_t, b1, w2_t, b2)


def _tc_jbond(jb0, jb1, w1_t, b1, w2_t, b2):
    def body(a_ref, b_ref, w1_ref, b1_ref, w2_ref, b2_ref, o_ref):
        h0 = _lrelu(
            jnp.dot(a_ref[...][:, 0:DIM], w1_ref[...],
                    preferred_element_type=jnp.float32) + b1_ref[...])
        h1 = _lrelu(
            jnp.dot(b_ref[...][:, 0:DIM], w1_ref[...],
                    preferred_element_type=jnp.float32) + b1_ref[...])
        p = (jnp.dot(h0, w2_ref[...], preferred_element_type=jnp.float32)
             + jnp.dot(h1, w2_ref[...], preferred_element_type=jnp.float32))
        o_ref[...] = 0.5 * p + b2_ref[...]

    return pl.pallas_call(
        body,
        out_shape=jax.ShapeDtypeStruct((N_JBOND, 1), jnp.float32),
    )(jb0, jb1, w1_t, b1, w2_t, b2)


def _tc_set2set(out_s, batch2, bih, bhh, out_w_t, out_b):
    def body(s_ref, b_ref, bih_ref, bhh_ref, ow_ref, ob_ref, o_ref):
        s_b = s_ref[...][:, 0:DIM]
        g = bih_ref[...] + bhh_ref[...]
        cc = jax.nn.sigmoid(g[:, 0:16]) * jnp.tanh(g[:, 32:48])
        q0 = jax.nn.sigmoid(g[:, 48:64]) * jnp.tanh(cc)
        e = jnp.sum(s_b * q0, axis=1, keepdims=True)
        a = jnp.exp(e - jnp.max(e))
        oh = (b_ref[...] == lax.broadcasted_iota(
            jnp.int32, (N_NODES, N_GRAPHS), 1)).astype(jnp.float32)
        w = oh * a
        ext = jnp.concatenate(
            [s_b, jnp.ones((N_NODES, 1), jnp.float32)], axis=1)
        rv = lax.dot_general(w, ext, (((0,), (0,)), ((), ())),
                             preferred_element_type=jnp.float32)
        rvec = rv[:, 0:16] / (rv[:, 16:17] + 1e-16)
        q_star = jnp.concatenate(
            [jnp.broadcast_to(q0, (N_GRAPHS, DIM)), rvec], axis=1)
        o_ref[...] = jnp.dot(q_star, ow_ref[...],
                             preferred_element_type=jnp.float32) + ob_ref[...]

    return pl.pallas_call(
        body,
        grid=(1,),
        in_specs=[
            pl.BlockSpec((N_NODES, LW), lambda i: (0, 0)),
            pl.BlockSpec((N_NODES, 1), lambda i: (0, 0)),
            pl.BlockSpec((1, 64), lambda i: (0, 0)),
            pl.BlockSpec((1, 64), lambda i: (0, 0)),
            pl.BlockSpec((32, 2), lambda i: (0, 0)),
            pl.BlockSpec((1, 2), lambda i: (0, 0)),
        ],
        out_specs=pl.BlockSpec((N_GRAPHS, 2), lambda i: (0, 0)),
        out_shape=jax.ShapeDtypeStruct((N_GRAPHS, 2), jnp.float32),
    )(out_s, batch2, bih, bhh, out_w_t, out_b)


def kernel(x, edge_attr, params, edge_index, stem_atmidx, jbond_atmidx, batch):
    p = params
    src, dst = edge_index[0], edge_index[1]

    e_extra = E_PAD - N_EDGES
    srcp = jnp.concatenate([src, jnp.zeros((e_extra,), jnp.int32)])
    dstp = jnp.concatenate([dst, jnp.full((e_extra,), N_NODES, jnp.int32)])
    eh = E_PAD // 2
    src3a = srcp[:eh].reshape(NW, KCH // 2, 128)
    src3b = srcp[eh:].reshape(NW, KCH // 2, 128)
    dst3a = dstp[:eh].reshape(NW, KCH // 2, 128)
    dst3b = dstp[eh:].reshape(NW, KCH // 2, 128)
    dst3 = dstp.reshape(NW, KCH, 128)
    sidx3 = jnp.concatenate([
        stem_atmidx, jbond_atmidx[:, 0], jbond_atmidx[:, 1],
        jnp.zeros((S_PAD - S_TOT,), jnp.int32)]).reshape(NW, SKCH, 128)
    xp = jnp.concatenate(
        [x, jnp.zeros((N_TBL - N_NODES, x.shape[1]), jnp.float32)], axis=0)
    ones_blk = jnp.ones((128, LW), jnp.float32)
    zeros_n = jnp.zeros((N_PAD, LW), jnp.float32)
    batch2 = batch.reshape(N_NODES, 1)

    w2 = p['edge_W2']
    ghat = w2.reshape(16, 16, 16).transpose(0, 2, 1).reshape(16, 256)
    b0 = p['edge_b2'].reshape(16, 16)
    eye = jnp.eye(16, dtype=jnp.float32)
    rm = jnp.repeat(eye, 16, axis=1)
    cm = jnp.tile(eye, (16, 1))
    lin0_wt = p['lin0_W'].T
    lin0_b = p['lin0_b'].reshape(1, DIM)
    e_w1t = p['edge_W1'].T
    e_b1 = p['edge_b1'].reshape(1, DIM)
    root = p['conv_root']
    conv_b = p['conv_bias'].reshape(1, DIM)
    wih_t = p['gru_Wih'].T
    whh_t = p['gru_Whh'].T
    bih = p['gru_bih'].reshape(1, 48)
    bhh = p['gru_bhh'].reshape(1, 48)
    st_w1t = p['stem_W1'].T
    st_b1 = p['stem_b1'].reshape(1, DIM)
    st_w2t = p['stem_W2'].T
    st_b2 = p['stem_b2'].reshape(1, 105)
    jb_w1t = p['jbond_W1'].T
    jb_b1 = p['jbond_b1'].reshape(1, DIM)
    jb_w2t = p['jbond_W2'].T
    jb_b2 = p['jbond_b2'].reshape(1, 1)
    l_bih = p['lstm_bih'].reshape(1, 64)
    l_bhh = p['lstm_bhh'].reshape(1, 64)
    out_wt = p['out_W'].T
    out_b = p['out_b'].reshape(1, 2)

    s = _tc_lin0(xp, lin0_wt, lin0_b)
    h1 = _tc_h1(edge_attr, e_w1t, e_b1)
    cnts = _sc_count(ones_blk, dst3, zeros_n)

    h1_half_blocks = eh // 4096
    for _ in range(6):
        xs_a = _sc_gather(s, src3a)
        msg_a = _tc_edge(xs_a, h1, ghat, b0, rm, cm, 0)
        xs_b = _sc_gather(s, src3b)
        parts_a = _sc_scatter_add(msg_a, dst3a, zeros_n)
        msg_b = _tc_edge(xs_b, h1, ghat, b0, rm, cm, h1_half_blocks)
        parts_b = _sc_scatter_add(msg_b, dst3b, zeros_n)
        s = _tc_node(s, parts_a, parts_b, cnts, root, conv_b, wih_t,
                     bih, whh_t, bhh)

    heads = _sc_gather(s, sidx3)
    ss = heads[:N_STEM]
    jb0 = heads[N_STEM:N_STEM + N_JBOND]
    jb1 = heads[N_STEM + N_JBOND:S_TOT]

    stem_preds = _tc_stem(ss, st_w1t, st_b1, st_w2t, st_b2)
    jbond_preds = _tc_jbond(jb0, jb1, jb_w1t, jb_b1, jb_w2t,
                            jb_b2).reshape(N_JBOND)
    res = _tc_set2set(s, batch2, l_bih, l_bhh, out_wt, out_b)
    return res, stem_preds, jbond_preds
